# baseline (speedup 1.0000x reference)
import jax
import jax.numpy as jnp
from jax import lax
from jax.experimental import pallas as pl
from jax.experimental.pallas import tpu as pltpu
from jax.experimental.pallas import tpu_sc as plsc

N = 10000
E = 160000
D = 128
H = 256
G = 64
HHALF = 128

NC = 2
NS = 16
ROWS_PER_TILE = N // NS
DCHUNK = 125
DROWS = E // DCHUNK
DCPT = DROWS // (NC * NS)
DWIN = 4
CHUNK = 80
VCPT = (E // CHUNK) // NS
CPT = 128
NQUAD = CPT // 4
GRP = 8
NGRP = CPT // GRP
RB = 1000
NRB = N // RB

_mesh = plsc.VectorSubcoreMesh(core_axis_name="c", subcore_axis_name="s",
                               num_cores=NC, num_subcores=NS)


def _deg_body(cidx_hbm, zeros_hbm, ones_hbm, out_hbm, acc, idxc_v, ones_v,
              isem, ssem):
    c = lax.axis_index("c")
    s = lax.axis_index("s")
    base = c * (NS * DCPT) + s * DCPT
    d1 = pltpu.async_copy(cidx_hbm.at[pl.ds(base, DCPT)], idxc_v, isem)
    d2 = pltpu.async_copy(ones_hbm, ones_v, isem)
    pltpu.sync_copy(zeros_hbm, acc.at[pl.ds(s * ROWS_PER_TILE, ROWS_PER_TILE)])
    d1.wait()
    d2.wait()
    plsc.subcore_barrier()

    def body(k, carry):
        pltpu.async_copy(ones_v, acc.at[idxc_v.at[k]], ssem, add=True)

        @pl.when(k >= DWIN)
        def _():
            pltpu.make_async_copy(ones_v, acc.at[idxc_v.at[0]], ssem).wait()

        return carry

    lax.fori_loop(0, DCPT, body, 0)
    for _ in range(DWIN):
        pltpu.make_async_copy(ones_v, acc.at[idxc_v.at[0]], ssem).wait()
    plsc.subcore_barrier()
    wid = c * NS + s
    pltpu.sync_copy(acc.at[pl.ds(s * ROWS_PER_TILE, ROWS_PER_TILE)],
                    out_hbm.at[wid])


_deg_call = pl.kernel(
    _deg_body,
    out_type=jax.ShapeDtypeStruct((NC * NS, ROWS_PER_TILE, 128), jnp.float32),
    mesh=_mesh,
    scratch_types=[
        pltpu.VMEM_SHARED((N, 128), jnp.float32),
        pltpu.VMEM((DCPT, DCHUNK), jnp.int32),
        pltpu.VMEM((DCHUNK, 128), jnp.float32),
        pltpu.SemaphoreType.DMA,
        pltpu.SemaphoreType.DMA,
    ],
)


def _scat_body(hflat_hbm, ridx_hbm, cidx_hbm, zeros_hbm, out_hbm,
               acc, idxr_v, idxc_v, buf0, buf1, buf2, buf3,
               gsem0, gsem1, gsem2, gsem3, ssem0, ssem1, ssem2, ssem3, isem):
    c = lax.axis_index("c")
    s = lax.axis_index("s")
    tbase = s * CPT
    gbufs = (buf0, buf1, buf2, buf3)
    gsems = (gsem0, gsem1, gsem2, gsem3)
    ssems = (ssem0, ssem1, ssem2, ssem3)

    def g_issue(q, ring_row, buf, sem):
        pltpu.async_copy(hflat_hbm.at[idxr_v.at[ring_row]], buf, sem)

    def g_wait(buf, sem):
        pltpu.make_async_copy(hflat_hbm.at[idxr_v.at[0]], buf, sem).wait()

    def s_issue(ring_row, buf, sem):
        pltpu.async_copy(buf, acc.at[idxc_v.at[ring_row]], sem, add=True)

    def s_wait(buf, sem):
        pltpu.make_async_copy(buf, acc.at[idxc_v.at[0]], sem).wait()

    d1 = pltpu.async_copy(ridx_hbm.at[c, pl.ds(tbase, GRP)],
                          idxr_v.at[pl.ds(0, GRP)], isem)
    d2 = pltpu.async_copy(cidx_hbm.at[pl.ds(tbase, GRP)],
                          idxc_v.at[pl.ds(0, GRP)], isem)
    pltpu.sync_copy(zeros_hbm, acc.at[pl.ds(s * ROWS_PER_TILE, ROWS_PER_TILE)])
    d1.wait()
    d2.wait()
    g_issue(0, 0, buf0, gsem0)
    g_issue(1, 1, buf1, gsem1)
    plsc.subcore_barrier()

    def body(k, carry):
        q0 = 4 * k
        r0 = lax.rem(q0, 2 * GRP)

        even = lax.rem(k, 2) == 0
        g = k // 2

        @pl.when(even & (g + 1 < NGRP))
        def _():
            half = lax.rem(g + 1, 2) * GRP
            srow = tbase + (g + 1) * GRP
            pltpu.async_copy(ridx_hbm.at[c, pl.ds(srow, GRP)],
                             idxr_v.at[pl.ds(half, GRP)], isem)
            pltpu.async_copy(cidx_hbm.at[pl.ds(srow, GRP)],
                             idxc_v.at[pl.ds(half, GRP)], isem)

        @pl.when(k > 0)
        def _():
            s_wait(buf2, ssem2)
            s_wait(buf3, ssem3)

        g_issue(q0 + 2, r0 + 2, buf2, gsem2)
        g_issue(q0 + 3, r0 + 3, buf3, gsem3)
        g_wait(buf0, gsem0)
        s_issue(r0, buf0, ssem0)
        g_wait(buf1, gsem1)
        s_issue(r0 + 1, buf1, ssem1)

        @pl.when((~even) & (g + 1 < NGRP))
        def _():
            pltpu.make_async_copy(cidx_hbm.at[pl.ds(0, GRP)],
                                  idxr_v.at[pl.ds(0, GRP)], isem).wait()
            pltpu.make_async_copy(cidx_hbm.at[pl.ds(0, GRP)],
                                  idxc_v.at[pl.ds(0, GRP)], isem).wait()

        s_wait(buf0, ssem0)
        s_wait(buf1, ssem1)

        @pl.when(k < NQUAD - 1)
        def _():
            rn = lax.rem(q0 + 4, 2 * GRP)
            g_issue(q0 + 4, rn, buf0, gsem0)
            g_issue(q0 + 5, rn + 1, buf1, gsem1)

        g_wait(buf2, gsem2)
        s_issue(r0 + 2, buf2, ssem2)
        g_wait(buf3, gsem3)
        s_issue(r0 + 3, buf3, ssem3)
        return carry

    lax.fori_loop(0, NQUAD, body, 0)
    s_wait(buf2, ssem2)
    s_wait(buf3, ssem3)
    plsc.subcore_barrier()
    wid = c * NS + s
    pltpu.sync_copy(acc.at[pl.ds(s * ROWS_PER_TILE, ROWS_PER_TILE)],
                    out_hbm.at[wid])


_scat_call = pl.kernel(
    _scat_body,
    out_type=jax.ShapeDtypeStruct((NC * NS, ROWS_PER_TILE, HHALF), jnp.float32),
    mesh=_mesh,
    scratch_types=[
        pltpu.VMEM_SHARED((N + 8, HHALF), jnp.float32),
        pltpu.VMEM((2 * GRP, CHUNK), jnp.int32),
        pltpu.VMEM((2 * GRP, CHUNK), jnp.int32),
        pltpu.VMEM((CHUNK, HHALF), jnp.float32),
        pltpu.VMEM((CHUNK, HHALF), jnp.float32),
        pltpu.VMEM((CHUNK, HHALF), jnp.float32),
        pltpu.VMEM((CHUNK, HHALF), jnp.float32),
        pltpu.SemaphoreType.DMA,
        pltpu.SemaphoreType.DMA,
        pltpu.SemaphoreType.DMA,
        pltpu.SemaphoreType.DMA,
        pltpu.SemaphoreType.DMA,
        pltpu.SemaphoreType.DMA,
        pltpu.SemaphoreType.DMA,
        pltpu.SemaphoreType.DMA,
        pltpu.SemaphoreType.DMA,
    ],
)


def _l1_body(x_ref, w_ref, dega_ref, degb_ref, out_ref):
    dinv = lax.rsqrt(dega_ref[0, 0, :] + degb_ref[0, 0, :] + 1.0)
    h = jnp.dot(x_ref[...], w_ref[...], preferred_element_type=jnp.float32)
    hp = h * dinv[:, None]
    out_ref[0, :, :] = hp[:, :HHALF]
    out_ref[1, :, :] = hp[:, HHALF:]


_l1_call = pl.pallas_call(
    _l1_body,
    grid=(NRB,),
    in_specs=[
        pl.BlockSpec((RB, D), lambda i: (i, 0)),
        pl.BlockSpec((D, H), lambda i: (0, 0)),
        pl.BlockSpec((1, 1, RB), lambda i: (i, 0, 0)),
        pl.BlockSpec((1, 1, RB), lambda i: (i, 0, 0)),
    ],
    out_specs=pl.BlockSpec((2, RB, HHALF), lambda i: (0, i, 0)),
    out_shape=jax.ShapeDtypeStruct((2, N, HHALF), jnp.float32),
)


def _l23_body(s_ref, hp_ref, dega_ref, degb_ref, b_ref, w_ref, out_ref):
    dinv = lax.rsqrt(dega_ref[0, 0, :] + degb_ref[0, 0, :] + 1.0)
    sfull = jnp.concatenate([s_ref[0], s_ref[1]], axis=-1)
    hpfull = jnp.concatenate([hp_ref[0], hp_ref[1]], axis=-1)
    xnew = jnp.maximum(dinv[:, None] * (sfull + hpfull) + b_ref[0], 0.0)
    h = jnp.dot(xnew, w_ref[...], preferred_element_type=jnp.float32)
    hp2 = h * dinv[:, None]
    out_ref[0, :, :] = hp2[:, :HHALF]
    out_ref[1, :, :] = hp2[:, HHALF:]


_l23_call = pl.pallas_call(
    _l23_body,
    grid=(NRB,),
    in_specs=[
        pl.BlockSpec((2, RB, HHALF), lambda i: (0, i, 0)),
        pl.BlockSpec((2, RB, HHALF), lambda i: (0, i, 0)),
        pl.BlockSpec((1, 1, RB), lambda i: (i, 0, 0)),
        pl.BlockSpec((1, 1, RB), lambda i: (i, 0, 0)),
        pl.BlockSpec((1, H), lambda i: (0, 0)),
        pl.BlockSpec((H, H), lambda i: (0, 0)),
    ],
    out_specs=pl.BlockSpec((2, RB, HHALF), lambda i: (0, i, 0)),
    out_shape=jax.ShapeDtypeStruct((2, N, HHALF), jnp.float32),
)


def _head_body(s_ref, hp_ref, dega_ref, degb_ref, b3_ref, batch_ref,
               wl1_ref, bl1_ref, wl_ref, bl_ref, out_ref, pool_acc, cnt_acc):
    i = pl.program_id(0)

    @pl.when(i == 0)
    def _():
        pool_acc[...] = jnp.zeros_like(pool_acc)
        cnt_acc[...] = jnp.zeros_like(cnt_acc)

    dinv = lax.rsqrt(dega_ref[0, 0, :] + degb_ref[0, 0, :] + 1.0)
    sfull = jnp.concatenate([s_ref[0], s_ref[1]], axis=-1)
    hpfull = jnp.concatenate([hp_ref[0], hp_ref[1]], axis=-1)
    x3 = jnp.maximum(dinv[:, None] * (sfull + hpfull) + b3_ref[0], 0.0)
    bb = batch_ref[0, 0, :]
    gids = lax.broadcasted_iota(jnp.int32, (G, RB), 0)
    P = (bb[None, :] == gids).astype(jnp.float32)
    pool_acc[...] += jnp.dot(P, x3, preferred_element_type=jnp.float32)
    cnt_acc[...] = cnt_acc[...] + jnp.sum(P, axis=1, keepdims=True)

    @pl.when(i == pl.num_programs(0) - 1)
    def _():
        cnt = cnt_acc[:, 0:1]
        pooled = pool_acc[...] / jnp.maximum(cnt, 1.0)
        z = jnp.maximum(
            jnp.dot(pooled, wl1_ref[...], preferred_element_type=jnp.float32)
            + bl1_ref[0], 0.0)
        out_ref[...] = (jnp.dot(z, wl_ref[...],
                                preferred_element_type=jnp.float32) + bl_ref[0])


_head_call = pl.pallas_call(
    _head_body,
    grid=(NRB,),
    in_specs=[
        pl.BlockSpec((2, RB, HHALF), lambda i: (0, i, 0)),
        pl.BlockSpec((2, RB, HHALF), lambda i: (0, i, 0)),
        pl.BlockSpec((1, 1, RB), lambda i: (i, 0, 0)),
        pl.BlockSpec((1, 1, RB), lambda i: (i, 0, 0)),
        pl.BlockSpec((1, H), lambda i: (0, 0)),
        pl.BlockSpec((1, 1, RB), lambda i: (i, 0, 0)),
        pl.BlockSpec((H, 32), lambda i: (0, 0)),
        pl.BlockSpec((1, 32), lambda i: (0, 0)),
        pl.BlockSpec((32, 2), lambda i: (0, 0)),
        pl.BlockSpec((1, 2), lambda i: (0, 0)),
    ],
    out_specs=pl.BlockSpec((G, 2), lambda i: (0, 0)),
    out_shape=jax.ShapeDtypeStruct((G, 2), jnp.float32),
    scratch_shapes=[
        pltpu.VMEM((G, H), jnp.float32),
        pltpu.VMEM((G, HHALF), jnp.float32),
    ],
)


def kernel(x, edge_index, batch, W1, b1, W2, b2, W3, b3, Wl1, bl1, Wl, bl):
    row = edge_index[0]
    col = edge_index[1]
    dcidx = col.reshape(DROWS, DCHUNK)
    rpad = jnp.concatenate(
        [row.reshape(NS, VCPT, CHUNK),
         jnp.zeros((NS, CPT - VCPT, CHUNK), jnp.int32)], axis=1
    ).reshape(NS * CPT, CHUNK)
    cpad = jnp.concatenate(
        [col.reshape(NS, VCPT, CHUNK),
         jnp.full((NS, CPT - VCPT, CHUNK), N, jnp.int32)], axis=1
    ).reshape(NS * CPT, CHUNK)
    ridx = jnp.stack([rpad, rpad + N])
    cidx = cpad
    zeros_slab = jnp.zeros((ROWS_PER_TILE, HHALF), jnp.float32)
    ones_chunk = jnp.ones((DCHUNK, 128), jnp.float32)

    deg_raw = _deg_call(dcidx, zeros_slab, ones_chunk)
    deg2 = deg_raw.reshape(NC, N, 128)[:, :, 0]
    deg_a = deg2[0].reshape(NRB, 1, RB)
    deg_b = deg2[1].reshape(NRB, 1, RB)

    hp = _l1_call(x, W1, deg_a, deg_b)
    for bprev, W in ((b1, W2), (b2, W3)):
        s_raw = _scat_call(hp.reshape(NC * N, HHALF), ridx, cidx, zeros_slab)
        s = s_raw.reshape(NC, N, HHALF)
        hp = _l23_call(s, hp, deg_a, deg_b, bprev.reshape(1, H), W)
    s_raw = _scat_call(hp.reshape(NC * N, HHALF), ridx, cidx, zeros_slab)
    s3 = s_raw.reshape(NC, N, HHALF)

    return _head_call(s3, hp, deg_a, deg_b, b3.reshape(1, H),
                      batch.reshape(NRB, 1, RB),
                      Wl1, bl1.reshape(1, 32), Wl, bl.reshape(1, 2))
